# baseline (device time: 17948 ns/iter reference)
import jax
import jax.numpy as jnp
from jax import lax
from jax.experimental import pallas as pl
from jax.experimental.pallas import tpu as pltpu

N_DEV = 4
PRE = 32
NGQ = 32


def kernel(x, Wq, K_ext, V_ext, Wo):
    B, Sql, E = x.shape
    _, Skl, Hq, Dh = K_ext.shape
    HD = Hq * Dh
    Skv = N_DEV * Skl
    PHD = HD + Hq

    def body(x_ref, wq_ref, k_ref, v_ref, wo_ref, out_ref,
             kfull, vfull, kst, vst, pkbuf, pvbuf, q0buf, paccbuf,
             ksend, krecv, vsend, vrecv, auxs, auxr, insem):
        my = lax.axis_index("i")
        left = lax.rem(my + (N_DEV - 1), N_DEV)
        right = lax.rem(my + 1, N_DEV)
        diag = lax.rem(my + 2, N_DEV)
        is0 = my == 0
        is2 = my == 2
        iseven = lax.rem(my, 2) == 0

        cp_k = pltpu.make_async_copy(k_ref, kst, insem.at[0])
        cp_v = pltpu.make_async_copy(v_ref, vst, insem.at[1])
        cp_k.start()
        cp_v.start()

        pkbuf[...] = jnp.zeros((B, PRE, HD), jnp.bfloat16)
        pvbuf[...] = jnp.zeros((B, PRE, HD), jnp.bfloat16)
        paccbuf[...] = jnp.zeros((B, NGQ, PHD), jnp.bfloat16)

        barrier = pltpu.get_barrier_semaphore()
        for nbr in (left, right):
            pltpu.semaphore_signal(
                barrier, inc=1,
                device_id=(nbr,), device_id_type=pl.DeviceIdType.MESH,
            )

        @pl.when(iseven)
        def _():
            pltpu.semaphore_signal(
                barrier, inc=1,
                device_id=(diag,), device_id_type=pl.DeviceIdType.MESH,
            )

        cp_k.wait()
        kfull[:, pl.ds(my * Skl, Skl), :] = (
            kst[...].astype(jnp.bfloat16).reshape(B, Skl, HD))
        cp_v.wait()
        vfull[:, pl.ds(my * Skl, Skl), :] = (
            vst[...].astype(jnp.bfloat16).reshape(B, Skl, HD))

        pltpu.semaphore_wait(barrier, 2)

        @pl.when(iseven)
        def _():
            pltpu.semaphore_wait(barrier, 1)

        sends = []
        for j, dest in enumerate((left, right)):
            for buf, ssem, rsem in ((kfull, ksend, krecv),
                                    (vfull, vsend, vrecv)):
                r = pltpu.make_async_remote_copy(
                    src_ref=buf.at[:, pl.ds(my * Skl, Skl), :],
                    dst_ref=buf.at[:, pl.ds(my * Skl, Skl), :],
                    send_sem=ssem.at[j], recv_sem=rsem.at[j],
                    device_id=(dest,), device_id_type=pl.DeviceIdType.MESH,
                )
                r.start()
                sends.append(r)

        @pl.when(is0)
        def _():
            for src, dst, i in ((kfull, pkbuf, 0), (vfull, pvbuf, 1)):
                r = pltpu.make_async_remote_copy(
                    src_ref=src.at[:, pl.ds(my * Skl, PRE), :],
                    dst_ref=dst,
                    send_sem=auxs.at[i], recv_sem=auxr.at[i],
                    device_id=(diag,), device_id_type=pl.DeviceIdType.MESH,
                )
                r.start()

        wq = wq_ref[...].astype(jnp.bfloat16)
        wo = wo_ref[...].astype(jnp.bfloat16)

        q = [
            (jnp.dot(x_ref[b].astype(jnp.bfloat16), wq,
                     preferred_element_type=jnp.float32)
             * 0.125).astype(jnp.bfloat16)
            for b in range(B)
        ]

        @pl.when(is0)
        def _():
            for b in range(B):
                q0buf[b] = q[b][:NGQ, :]
            r = pltpu.make_async_remote_copy(
                src_ref=q0buf, dst_ref=q0buf,
                send_sem=auxs.at[2], recv_sem=auxr.at[2],
                device_id=(diag,), device_id_type=pl.DeviceIdType.MESH,
            )
            r.start()

        qrow = lax.broadcasted_iota(jnp.int32, (Sql, Skl), 0) + my * Sql
        kcol = lax.broadcasted_iota(jnp.int32, (Sql, Skl), 1)

        acc = [[jnp.zeros((Sql, Dh), jnp.float32) for _ in range(Hq)]
               for _ in range(B)]
        lsum = [[jnp.zeros((Sql, 1), jnp.float32) for _ in range(Hq)]
                for _ in range(B)]

        def process_block(origin):
            ki = kcol + origin * Skl
            mask = (jnp.abs(qrow - ki) <= 128) | (ki < 32) | (qrow < 32)
            for b in range(B):
                kb = kfull[b, pl.ds(origin * Skl, Skl), :]
                vb = vfull[b, pl.ds(origin * Skl, Skl), :]
                for h in range(Hq):
                    qh = q[b][:, h * Dh:(h + 1) * Dh]
                    kh = kb[:, h * Dh:(h + 1) * Dh]
                    s = lax.dot_general(
                        qh, kh, (((1,), (1,)), ((), ())),
                        preferred_element_type=jnp.float32,
                    )
                    p = jnp.exp(jnp.where(mask, s, -1e9))
                    lsum[b][h] = lsum[b][h] + jnp.sum(p, axis=-1,
                                                      keepdims=True)
                    acc[b][h] = acc[b][h] + jnp.dot(
                        p.astype(jnp.bfloat16), vb[:, h * Dh:(h + 1) * Dh],
                        preferred_element_type=jnp.float32,
                    )

        process_block(my)

        @pl.when(is2)
        def _():
            rq = pltpu.make_async_remote_copy(
                src_ref=q0buf, dst_ref=q0buf,
                send_sem=auxs.at[2], recv_sem=auxr.at[2],
                device_id=(diag,), device_id_type=pl.DeviceIdType.MESH,
            )
            rq.wait_recv()
            for b in range(B):
                q0 = q0buf[b]
                kb = kfull[b, pl.ds(my * Skl, Skl), :]
                vb = vfull[b, pl.ds(my * Skl, Skl), :]
                pieces, lsums = [], []
                for h in range(Hq):
                    s = lax.dot_general(
                        q0[:, h * Dh:(h + 1) * Dh],
                        kb[:, h * Dh:(h + 1) * Dh],
                        (((1,), (1,)), ((), ())),
                        preferred_element_type=jnp.float32,
                    )
                    p = jnp.exp(s)
                    pieces.append(jnp.dot(
                        p.astype(jnp.bfloat16), vb[:, h * Dh:(h + 1) * Dh],
                        preferred_element_type=jnp.float32,
                    ).astype(jnp.bfloat16))
                    lsums.append(jnp.sum(p, axis=-1, keepdims=True)
                                 .astype(jnp.bfloat16))
                paccbuf[b] = jnp.concatenate(pieces + lsums, axis=1)
            rp = pltpu.make_async_remote_copy(
                src_ref=paccbuf, dst_ref=paccbuf,
                send_sem=auxs.at[3], recv_sem=auxr.at[3],
                device_id=(diag,), device_id_type=pl.DeviceIdType.MESH,
            )
            rp.start()

        for j, origin in enumerate((right, left)):
            recv_k = pltpu.make_async_remote_copy(
                src_ref=kfull.at[:, pl.ds(origin * Skl, Skl), :],
                dst_ref=kfull.at[:, pl.ds(origin * Skl, Skl), :],
                send_sem=ksend.at[j], recv_sem=krecv.at[j],
                device_id=(origin,), device_id_type=pl.DeviceIdType.MESH,
            )
            recv_v = pltpu.make_async_remote_copy(
                src_ref=vfull.at[:, pl.ds(origin * Skl, Skl), :],
                dst_ref=vfull.at[:, pl.ds(origin * Skl, Skl), :],
                send_sem=vsend.at[j], recv_sem=vrecv.at[j],
                device_id=(origin,), device_id_type=pl.DeviceIdType.MESH,
            )
            recv_k.wait_recv()
            recv_v.wait_recv()
            process_block(origin)

        @pl.when(is2)
        def _():
            for i in range(2):
                r = pltpu.make_async_remote_copy(
                    src_ref=pkbuf, dst_ref=(pkbuf, pvbuf)[i],
                    send_sem=auxs.at[i], recv_sem=auxr.at[i],
                    device_id=(diag,), device_id_type=pl.DeviceIdType.MESH,
                )
                r.wait_recv()
        for b in range(B):
            pk = pkbuf[b]
            pv = pvbuf[b]
            for h in range(Hq):
                s = lax.dot_general(
                    q[b][:, h * Dh:(h + 1) * Dh],
                    pk[:, h * Dh:(h + 1) * Dh],
                    (((1,), (1,)), ((), ())),
                    preferred_element_type=jnp.float32,
                )
                p = jnp.where(is2, jnp.exp(s), 0.0)
                lsum[b][h] = lsum[b][h] + jnp.sum(p, axis=-1, keepdims=True)
                acc[b][h] = acc[b][h] + jnp.dot(
                    p.astype(jnp.bfloat16), pv[:, h * Dh:(h + 1) * Dh],
                    preferred_element_type=jnp.float32,
                )

        @pl.when(is0)
        def _():
            rp = pltpu.make_async_remote_copy(
                src_ref=paccbuf, dst_ref=paccbuf,
                send_sem=auxs.at[3], recv_sem=auxr.at[3],
                device_id=(diag,), device_id_type=pl.DeviceIdType.MESH,
            )
            rp.wait_recv()

        zrow_a = jnp.zeros((Sql - NGQ, Dh), jnp.float32)
        zrow_l = jnp.zeros((Sql - NGQ, 1), jnp.float32)
        for b in range(B):
            cols = []
            for h in range(Hq):
                pacc = paccbuf[b][:, h * Dh:(h + 1) * Dh].astype(jnp.float32)
                plsum = paccbuf[b][:, HD + h:HD + h + 1].astype(jnp.float32)
                a = acc[b][h] + jnp.where(
                    is0, jnp.concatenate([pacc, zrow_a], axis=0), 0.0)
                l = lsum[b][h] + jnp.where(
                    is0, jnp.concatenate([plsum, zrow_l], axis=0), 0.0)
                cols.append((a / l).astype(jnp.bfloat16))
            ctx = jnp.concatenate(cols, axis=1)
            out_ref[b] = jnp.dot(ctx, wo,
                                 preferred_element_type=jnp.float32)

        for r in sends:
            r.wait_send()

        @pl.when(is0)
        def _():
            for i, dst in ((0, pkbuf), (1, pvbuf), (2, q0buf)):
                src = (kfull.at[:, pl.ds(my * Skl, PRE), :] if i == 0 else
                       vfull.at[:, pl.ds(my * Skl, PRE), :] if i == 1 else
                       q0buf)
                r = pltpu.make_async_remote_copy(
                    src_ref=src, dst_ref=dst,
                    send_sem=auxs.at[i], recv_sem=auxr.at[i],
                    device_id=(diag,), device_id_type=pl.DeviceIdType.MESH,
                )
                r.wait_send()

        @pl.when(is2)
        def _():
            r = pltpu.make_async_remote_copy(
                src_ref=paccbuf, dst_ref=paccbuf,
                send_sem=auxs.at[3], recv_sem=auxr.at[3],
                device_id=(diag,), device_id_type=pl.DeviceIdType.MESH,
            )
            r.wait_send()

    return pl.pallas_call(
        body,
        out_shape=jax.ShapeDtypeStruct((B, Sql, E), jnp.float32),
        in_specs=[
            pl.BlockSpec(memory_space=pltpu.VMEM),
            pl.BlockSpec(memory_space=pltpu.VMEM),
            pl.BlockSpec(memory_space=pl.ANY),
            pl.BlockSpec(memory_space=pl.ANY),
            pl.BlockSpec(memory_space=pltpu.VMEM),
        ],
        out_specs=pl.BlockSpec(memory_space=pltpu.VMEM),
        scratch_shapes=[
            pltpu.VMEM((B, Skv, HD), jnp.bfloat16),
            pltpu.VMEM((B, Skv, HD), jnp.bfloat16),
            pltpu.VMEM((B, Skl, Hq, Dh), jnp.float32),
            pltpu.VMEM((B, Skl, Hq, Dh), jnp.float32),
            pltpu.VMEM((B, PRE, HD), jnp.bfloat16),
            pltpu.VMEM((B, PRE, HD), jnp.bfloat16),
            pltpu.VMEM((B, NGQ, HD), jnp.bfloat16),
            pltpu.VMEM((B, NGQ, PHD), jnp.bfloat16),
            pltpu.SemaphoreType.DMA((2,)),
            pltpu.SemaphoreType.DMA((2,)),
            pltpu.SemaphoreType.DMA((2,)),
            pltpu.SemaphoreType.DMA((2,)),
            pltpu.SemaphoreType.DMA((4,)),
            pltpu.SemaphoreType.DMA((4,)),
            pltpu.SemaphoreType.DMA((2,)),
        ],
        compiler_params=pltpu.CompilerParams(collective_id=0),
    )(x, Wq, K_ext, V_ext, Wo)


# device time: 15980 ns/iter; 1.1232x vs baseline; 1.1232x over previous
import jax
import jax.numpy as jnp
from jax import lax
from jax.experimental import pallas as pl
from jax.experimental.pallas import tpu as pltpu

N_DEV = 4
PRE = 32
NGQ = 32


def kernel(x, Wq, K_ext, V_ext, Wo):
    B, Sql, E = x.shape
    _, Skl, Hq, Dh = K_ext.shape
    HD = Hq * Dh
    Skv = N_DEV * Skl
    PHD = HD + Hq

    def body(x_ref, wq_ref, k_ref, v_ref, wo_ref, out_ref,
             kfull, vfull, kst, vst, pkbuf, pvbuf, q0buf, paccbuf,
             ksend, krecv, vsend, vrecv, auxs, auxr, acksem, insem):
        my = lax.axis_index("i")
        left = lax.rem(my + (N_DEV - 1), N_DEV)
        right = lax.rem(my + 1, N_DEV)
        diag = lax.rem(my + 2, N_DEV)
        is0 = my == 0
        is2 = my == 2
        is3 = my == 3
        iseven = lax.rem(my, 2) == 0

        cp_k = pltpu.make_async_copy(k_ref, kst, insem.at[0])
        cp_v = pltpu.make_async_copy(v_ref, vst, insem.at[1])
        cp_k.start()
        cp_v.start()

        kfull[...] = jnp.zeros((B, Skv, HD), jnp.bfloat16)
        vfull[...] = jnp.zeros((B, Skv, HD), jnp.bfloat16)
        pkbuf[...] = jnp.zeros((B, PRE, HD), jnp.bfloat16)
        pvbuf[...] = jnp.zeros((B, PRE, HD), jnp.bfloat16)
        paccbuf[...] = jnp.zeros((B, NGQ, PHD), jnp.bfloat16)

        barrier = pltpu.get_barrier_semaphore()
        for nbr in (left, right):
            pltpu.semaphore_signal(
                barrier, inc=1,
                device_id=(nbr,), device_id_type=pl.DeviceIdType.MESH,
            )

        @pl.when(iseven)
        def _():
            pltpu.semaphore_signal(
                barrier, inc=1,
                device_id=(diag,), device_id_type=pl.DeviceIdType.MESH,
            )

        cp_k.wait()
        kfull[:, pl.ds(my * Skl, Skl), :] = (
            kst[...].astype(jnp.bfloat16).reshape(B, Skl, HD))
        cp_v.wait()
        vfull[:, pl.ds(my * Skl, Skl), :] = (
            vst[...].astype(jnp.bfloat16).reshape(B, Skl, HD))

        pltpu.semaphore_wait(barrier, 2)

        @pl.when(iseven)
        def _():
            pltpu.semaphore_wait(barrier, 1)

        def block_rdma(j, dest, origin):
            out = []
            for buf, ssem, rsem in ((kfull, ksend, krecv),
                                    (vfull, vsend, vrecv)):
                out.append(pltpu.make_async_remote_copy(
                    src_ref=buf.at[:, pl.ds(origin * Skl, Skl), :],
                    dst_ref=buf.at[:, pl.ds(origin * Skl, Skl), :],
                    send_sem=ssem.at[j], recv_sem=rsem.at[j],
                    device_id=(dest,), device_id_type=pl.DeviceIdType.MESH,
                ))
            return out

        @pl.when(my != 0)
        def _():
            for r in block_rdma(0, left, my):
                r.start()

        for r in block_rdma(1, right, my):
            r.start()

        def prefix_rdma(i, src, dst, dest):
            return pltpu.make_async_remote_copy(
                src_ref=src.at[:, pl.ds(my * Skl, PRE), :],
                dst_ref=dst,
                send_sem=auxs.at[i], recv_sem=auxr.at[i],
                device_id=(dest,), device_id_type=pl.DeviceIdType.MESH,
            )

        @pl.when(is0)
        def _():
            prefix_rdma(0, kfull, pkbuf, diag).start()
            prefix_rdma(1, vfull, pvbuf, diag).start()
            prefix_rdma(4, kfull, pkbuf, left).start()
            prefix_rdma(5, vfull, pvbuf, left).start()

        wq = wq_ref[...].astype(jnp.bfloat16)
        wo = wo_ref[...].astype(jnp.bfloat16)

        q = [
            (jnp.dot(x_ref[b].astype(jnp.bfloat16), wq,
                     preferred_element_type=jnp.float32)
             * 0.125).astype(jnp.bfloat16)
            for b in range(B)
        ]

        def q0_rdma():
            return pltpu.make_async_remote_copy(
                src_ref=q0buf, dst_ref=q0buf,
                send_sem=auxs.at[2], recv_sem=auxr.at[2],
                device_id=(left,), device_id_type=pl.DeviceIdType.MESH,
            )

        @pl.when(is0)
        def _():
            for b in range(B):
                q0buf[b] = q[b][:NGQ, :]
            q0_rdma().start()

        qrow = lax.broadcasted_iota(jnp.int32, (Sql, Skl), 0) + my * Sql
        kcol = lax.broadcasted_iota(jnp.int32, (Sql, Skl), 1)

        acc = [[jnp.zeros((Sql, Dh), jnp.float32) for _ in range(Hq)]
               for _ in range(B)]
        lsum = [[jnp.zeros((Sql, 1), jnp.float32) for _ in range(Hq)]
                for _ in range(B)]

        def process_block(origin, enable=None):
            ki = kcol + origin * Skl
            mask = (jnp.abs(qrow - ki) <= 128) | (ki < 32) | (qrow < 32)
            if enable is not None:
                mask = mask & enable
            for b in range(B):
                kb = kfull[b, pl.ds(origin * Skl, Skl), :]
                vb = vfull[b, pl.ds(origin * Skl, Skl), :]
                for h in range(Hq):
                    qh = q[b][:, h * Dh:(h + 1) * Dh]
                    kh = kb[:, h * Dh:(h + 1) * Dh]
                    s = lax.dot_general(
                        qh, kh, (((1,), (1,)), ((), ())),
                        preferred_element_type=jnp.float32,
                    )
                    p = jnp.exp(jnp.where(mask, s, -1e9))
                    lsum[b][h] = lsum[b][h] + jnp.sum(p, axis=-1,
                                                      keepdims=True)
                    acc[b][h] = acc[b][h] + jnp.dot(
                        p.astype(jnp.bfloat16), vb[:, h * Dh:(h + 1) * Dh],
                        preferred_element_type=jnp.float32,
                    )

        process_block(my)

        @pl.when(my != 3)
        def _():
            for r in block_rdma(0, right, right):
                r.wait_recv()
        process_block(right, enable=my != 3)

        for r in block_rdma(1, left, left):
            r.wait_recv()
        process_block(left)

        @pl.when(is3)
        def _():
            q0_rdma().wait_recv()
            for b in range(B):
                q0 = q0buf[b]
                kb = kfull[b, pl.ds(left * Skl, Skl), :]
                vb = vfull[b, pl.ds(left * Skl, Skl), :]
                pieces, lsums = [], []
                for h in range(Hq):
                    s = lax.dot_general(
                        q0[:, h * Dh:(h + 1) * Dh],
                        kb[:, h * Dh:(h + 1) * Dh],
                        (((1,), (1,)), ((), ())),
                        preferred_element_type=jnp.float32,
                    )
                    p = jnp.exp(s)
                    pieces.append(jnp.dot(
                        p.astype(jnp.bfloat16), vb[:, h * Dh:(h + 1) * Dh],
                        preferred_element_type=jnp.float32,
                    ).astype(jnp.bfloat16))
                    lsums.append(jnp.sum(p, axis=-1, keepdims=True)
                                 .astype(jnp.bfloat16))
                paccbuf[b] = jnp.concatenate(pieces + lsums, axis=1)
            pltpu.make_async_remote_copy(
                src_ref=paccbuf, dst_ref=paccbuf,
                send_sem=auxs.at[3], recv_sem=auxr.at[3],
                device_id=(right,), device_id_type=pl.DeviceIdType.MESH,
            ).start()

        @pl.when(is2)
        def _():
            prefix_rdma(0, kfull, pkbuf, diag).wait_recv()
            prefix_rdma(1, vfull, pvbuf, diag).wait_recv()
            pltpu.semaphore_signal(
                acksem, inc=1,
                device_id=(diag,), device_id_type=pl.DeviceIdType.MESH,
            )

        @pl.when(is3)
        def _():
            prefix_rdma(4, kfull, pkbuf, right).wait_recv()
            prefix_rdma(5, vfull, pvbuf, right).wait_recv()

        pref_en = is2 | is3
        for b in range(B):
            pk = pkbuf[b]
            pv = pvbuf[b]
            for h in range(Hq):
                s = lax.dot_general(
                    q[b][:, h * Dh:(h + 1) * Dh],
                    pk[:, h * Dh:(h + 1) * Dh],
                    (((1,), (1,)), ((), ())),
                    preferred_element_type=jnp.float32,
                )
                p = jnp.where(pref_en, jnp.exp(s), 0.0)
                lsum[b][h] = lsum[b][h] + jnp.sum(p, axis=-1, keepdims=True)
                acc[b][h] = acc[b][h] + jnp.dot(
                    p.astype(jnp.bfloat16), pv[:, h * Dh:(h + 1) * Dh],
                    preferred_element_type=jnp.float32,
                )

        @pl.when(is0)
        def _():
            pltpu.make_async_remote_copy(
                src_ref=paccbuf, dst_ref=paccbuf,
                send_sem=auxs.at[3], recv_sem=auxr.at[3],
                device_id=(left,), device_id_type=pl.DeviceIdType.MESH,
            ).wait_recv()

        zrow_a = jnp.zeros((Sql - NGQ, Dh), jnp.float32)
        zrow_l = jnp.zeros((Sql - NGQ, 1), jnp.float32)
        for b in range(B):
            cols = []
            for h in range(Hq):
                pacc = paccbuf[b][:, h * Dh:(h + 1) * Dh].astype(jnp.float32)
                plsum = paccbuf[b][:, HD + h:HD + h + 1].astype(jnp.float32)
                a = acc[b][h] + jnp.where(
                    is0, jnp.concatenate([pacc, zrow_a], axis=0), 0.0)
                l = lsum[b][h] + jnp.where(
                    is0, jnp.concatenate([plsum, zrow_l], axis=0), 0.0)
                cols.append((a / l).astype(jnp.bfloat16))
            ctx = jnp.concatenate(cols, axis=1)
            out_ref[b] = jnp.dot(ctx, wo,
                                 preferred_element_type=jnp.float32)

        @pl.when(my != 0)
        def _():
            for r in block_rdma(0, left, my):
                r.wait_send()

        for r in block_rdma(1, right, my):
            r.wait_send()

        @pl.when(is0)
        def _():
            prefix_rdma(0, kfull, pkbuf, diag).wait_send()
            prefix_rdma(1, vfull, pvbuf, diag).wait_send()
            prefix_rdma(4, kfull, pkbuf, left).wait_send()
            prefix_rdma(5, vfull, pvbuf, left).wait_send()
            q0_rdma().wait_send()
            pltpu.semaphore_wait(acksem, 1)

        @pl.when(is3)
        def _():
            pltpu.make_async_remote_copy(
                src_ref=paccbuf, dst_ref=paccbuf,
                send_sem=auxs.at[3], recv_sem=auxr.at[3],
                device_id=(right,), device_id_type=pl.DeviceIdType.MESH,
            ).wait_send()

    return pl.pallas_call(
        body,
        out_shape=jax.ShapeDtypeStruct((B, Sql, E), jnp.float32),
        in_specs=[
            pl.BlockSpec(memory_space=pltpu.VMEM),
            pl.BlockSpec(memory_space=pltpu.VMEM),
            pl.BlockSpec(memory_space=pl.ANY),
            pl.BlockSpec(memory_space=pl.ANY),
            pl.BlockSpec(memory_space=pltpu.VMEM),
        ],
        out_specs=pl.BlockSpec(memory_space=pltpu.VMEM),
        scratch_shapes=[
            pltpu.VMEM((B, Skv, HD), jnp.bfloat16),
            pltpu.VMEM((B, Skv, HD), jnp.bfloat16),
            pltpu.VMEM((B, Skl, Hq, Dh), jnp.float32),
            pltpu.VMEM((B, Skl, Hq, Dh), jnp.float32),
            pltpu.VMEM((B, PRE, HD), jnp.bfloat16),
            pltpu.VMEM((B, PRE, HD), jnp.bfloat16),
            pltpu.VMEM((B, NGQ, HD), jnp.bfloat16),
            pltpu.VMEM((B, NGQ, PHD), jnp.bfloat16),
            pltpu.SemaphoreType.DMA((2,)),
            pltpu.SemaphoreType.DMA((2,)),
            pltpu.SemaphoreType.DMA((2,)),
            pltpu.SemaphoreType.DMA((2,)),
            pltpu.SemaphoreType.DMA((6,)),
            pltpu.SemaphoreType.DMA((6,)),
            pltpu.SemaphoreType.REGULAR,
            pltpu.SemaphoreType.DMA((2,)),
        ],
        compiler_params=pltpu.CompilerParams(collective_id=0),
    )(x, Wq, K_ext, V_ext, Wo)


# device time: 13797 ns/iter; 1.3009x vs baseline; 1.1582x over previous
import jax
import jax.numpy as jnp
from jax import lax
from jax.experimental import pallas as pl
from jax.experimental.pallas import tpu as pltpu

N_DEV = 4


def kernel(x, Wq, K_ext, V_ext, Wo):
    B, Sql, E = x.shape
    _, Skl, Hq, Dh = K_ext.shape
    HD = Hq * Dh
    Skv = N_DEV * Skl

    def body(x_ref, wq_ref, k_ref, v_ref, wo_ref, out_ref,
             kfull, vfull, kst, vst, ksend, krecv, vsend, vrecv, insem):
        my = lax.axis_index("i")
        left = lax.rem(my + (N_DEV - 1), N_DEV)
        right = lax.rem(my + 1, N_DEV)
        diag = lax.rem(my + 2, N_DEV)
        peers = (left, right, diag)

        cp_k = pltpu.make_async_copy(k_ref, kst, insem.at[0])
        cp_v = pltpu.make_async_copy(v_ref, vst, insem.at[1])
        cp_k.start()
        cp_v.start()

        barrier = pltpu.get_barrier_semaphore()
        for nbr in peers:
            pltpu.semaphore_signal(
                barrier, inc=1,
                device_id=(nbr,), device_id_type=pl.DeviceIdType.MESH,
            )

        cp_k.wait()
        kfull[:, pl.ds(my * Skl, Skl), :] = (
            kst[...].astype(jnp.float8_e4m3fn).reshape(B, Skl, HD))
        cp_v.wait()
        vfull[:, pl.ds(my * Skl, Skl), :] = (
            vst[...].astype(jnp.bfloat16).reshape(B, Skl, HD))

        pltpu.semaphore_wait(barrier, len(peers))

        sends = []
        for j, dest in enumerate(peers):
            for buf, ssem, rsem in ((kfull, ksend, krecv),
                                    (vfull, vsend, vrecv)):
                r = pltpu.make_async_remote_copy(
                    src_ref=buf.at[:, pl.ds(my * Skl, Skl), :],
                    dst_ref=buf.at[:, pl.ds(my * Skl, Skl), :],
                    send_sem=ssem.at[j], recv_sem=rsem.at[j],
                    device_id=(dest,), device_id_type=pl.DeviceIdType.MESH,
                )
                r.start()
                sends.append(r)

        wq = wq_ref[...].astype(jnp.bfloat16)
        wo = wo_ref[...].astype(jnp.bfloat16)

        q = [
            (jnp.dot(x_ref[b].astype(jnp.bfloat16), wq,
                     preferred_element_type=jnp.float32)
             * 0.125).astype(jnp.bfloat16)
            for b in range(B)
        ]

        qrow = lax.broadcasted_iota(jnp.int32, (Sql, Skl), 0) + my * Sql
        kcol = lax.broadcasted_iota(jnp.int32, (Sql, Skl), 1)

        acc = [[jnp.zeros((Sql, Dh), jnp.float32) for _ in range(Hq)]
               for _ in range(B)]
        lsum = [[jnp.zeros((Sql, 1), jnp.float32) for _ in range(Hq)]
                for _ in range(B)]

        def process_block(origin):
            ki = kcol + origin * Skl
            mask = (jnp.abs(qrow - ki) <= 128) | (ki < 32) | (qrow < 32)
            for b in range(B):
                kb = kfull[b, pl.ds(origin * Skl, Skl), :]
                vb = vfull[b, pl.ds(origin * Skl, Skl), :]
                for h in range(Hq):
                    qh = q[b][:, h * Dh:(h + 1) * Dh]
                    kh = kb[:, h * Dh:(h + 1) * Dh].astype(jnp.bfloat16)
                    s = lax.dot_general(
                        qh, kh, (((1,), (1,)), ((), ())),
                        preferred_element_type=jnp.float32,
                    )
                    p = jnp.exp(jnp.where(mask, s, -1e9))
                    lsum[b][h] = lsum[b][h] + jnp.sum(p, axis=-1,
                                                      keepdims=True)
                    acc[b][h] = acc[b][h] + jnp.dot(
                        p.astype(jnp.bfloat16),
                        vb[:, h * Dh:(h + 1) * Dh],
                        preferred_element_type=jnp.float32,
                    )

        process_block(my)
        for j, origin in enumerate((right, left, diag)):
            recv_k = pltpu.make_async_remote_copy(
                src_ref=kfull.at[:, pl.ds(origin * Skl, Skl), :],
                dst_ref=kfull.at[:, pl.ds(origin * Skl, Skl), :],
                send_sem=ksend.at[j], recv_sem=krecv.at[j],
                device_id=(origin,), device_id_type=pl.DeviceIdType.MESH,
            )
            recv_v = pltpu.make_async_remote_copy(
                src_ref=vfull.at[:, pl.ds(origin * Skl, Skl), :],
                dst_ref=vfull.at[:, pl.ds(origin * Skl, Skl), :],
                send_sem=vsend.at[j], recv_sem=vrecv.at[j],
                device_id=(origin,), device_id_type=pl.DeviceIdType.MESH,
            )
            recv_k.wait_recv()
            recv_v.wait_recv()
            process_block(origin)

        for b in range(B):
            ctx = jnp.concatenate(
                [(acc[b][h] / lsum[b][h]).astype(jnp.bfloat16)
                 for h in range(Hq)],
                axis=1,
            )
            out_ref[b] = jnp.dot(ctx, wo,
                                 preferred_element_type=jnp.float32)

        for r in sends:
            r.wait_send()

    return pl.pallas_call(
        body,
        out_shape=jax.ShapeDtypeStruct((B, Sql, E), jnp.float32),
        in_specs=[
            pl.BlockSpec(memory_space=pltpu.VMEM),
            pl.BlockSpec(memory_space=pltpu.VMEM),
            pl.BlockSpec(memory_space=pl.ANY),
            pl.BlockSpec(memory_space=pl.ANY),
            pl.BlockSpec(memory_space=pltpu.VMEM),
        ],
        out_specs=pl.BlockSpec(memory_space=pltpu.VMEM),
        scratch_shapes=[
            pltpu.VMEM((B, Skv, HD), jnp.float8_e4m3fn),
            pltpu.VMEM((B, Skv, HD), jnp.bfloat16),
            pltpu.VMEM((B, Skl, Hq, Dh), jnp.float32),
            pltpu.VMEM((B, Skl, Hq, Dh), jnp.float32),
            pltpu.SemaphoreType.DMA((N_DEV - 1,)),
            pltpu.SemaphoreType.DMA((N_DEV - 1,)),
            pltpu.SemaphoreType.DMA((N_DEV - 1,)),
            pltpu.SemaphoreType.DMA((N_DEV - 1,)),
            pltpu.SemaphoreType.DMA((2,)),
        ],
        compiler_params=pltpu.CompilerParams(collective_id=0),
    )(x, Wq, K_ext, V_ext, Wo)


# device time: 12942 ns/iter; 1.3868x vs baseline; 1.0661x over previous
import jax
import jax.numpy as jnp
from jax import lax
from jax.experimental import pallas as pl
from jax.experimental.pallas import tpu as pltpu

N_DEV = 4


def kernel(x, Wq, K_ext, V_ext, Wo):
    B, Sql, E = x.shape
    _, Skl, Hq, Dh = K_ext.shape
    HD = Hq * Dh
    Skv = N_DEV * Skl

    def body(x_ref, wq_ref, k_ref, v_ref, wo_ref, out_ref,
             kfull, vfull, ksc, vsc, kst, vst,
             ksend, krecv, vsend, vrecv, ssend, srecv, insem):
        my = lax.axis_index("i")
        left = lax.rem(my + (N_DEV - 1), N_DEV)
        right = lax.rem(my + 1, N_DEV)
        diag = lax.rem(my + 2, N_DEV)
        peers = (left, right, diag)

        cp_k = pltpu.make_async_copy(k_ref, kst, insem.at[0])
        cp_v = pltpu.make_async_copy(v_ref, vst, insem.at[1])
        cp_k.start()
        cp_v.start()

        barrier = pltpu.get_barrier_semaphore()
        for nbr in peers:
            pltpu.semaphore_signal(
                barrier, inc=1,
                device_id=(nbr,), device_id_type=pl.DeviceIdType.MESH,
            )

        def quantize(st_ref, qfull, qsc, cp):
            cp.wait()
            val = st_ref[...].reshape(B, Skl, HD)
            amax = jnp.maximum(jnp.max(jnp.abs(val), axis=-1), 1e-20)
            scale = amax * (1.0 / 127.0)
            qfull[:, pl.ds(my * Skl, Skl), :] = jnp.round(
                val / scale[:, :, None]).astype(jnp.int8)
            qsc[:, pl.ds(my * Skl, Skl)] = scale

        quantize(kst, kfull, ksc, cp_k)
        quantize(vst, vfull, vsc, cp_v)

        pltpu.semaphore_wait(barrier, len(peers))

        sends = []
        for j, dest in enumerate(peers):
            for buf, ssm, rsm in ((kfull, ksend, krecv),
                                  (vfull, vsend, vrecv)):
                r = pltpu.make_async_remote_copy(
                    src_ref=buf.at[:, pl.ds(my * Skl, Skl), :],
                    dst_ref=buf.at[:, pl.ds(my * Skl, Skl), :],
                    send_sem=ssm.at[j], recv_sem=rsm.at[j],
                    device_id=(dest,), device_id_type=pl.DeviceIdType.MESH,
                )
                r.start()
                sends.append(r)
            for i, buf in enumerate((ksc, vsc)):
                r = pltpu.make_async_remote_copy(
                    src_ref=buf.at[:, pl.ds(my * Skl, Skl)],
                    dst_ref=buf.at[:, pl.ds(my * Skl, Skl)],
                    send_sem=ssend.at[i, j], recv_sem=srecv.at[i, j],
                    device_id=(dest,), device_id_type=pl.DeviceIdType.MESH,
                )
                r.start()
                sends.append(r)

        wq = wq_ref[...].astype(jnp.bfloat16)
        wo = wo_ref[...].astype(jnp.bfloat16)

        q = [
            (jnp.dot(x_ref[b].astype(jnp.bfloat16), wq,
                     preferred_element_type=jnp.float32)
             * 0.125).astype(jnp.bfloat16)
            for b in range(B)
        ]

        qrow = lax.broadcasted_iota(jnp.int32, (Sql, Skl), 0) + my * Sql
        kcol = lax.broadcasted_iota(jnp.int32, (Sql, Skl), 1)

        acc = [[jnp.zeros((Sql, Dh), jnp.float32) for _ in range(Hq)]
               for _ in range(B)]
        lsum = [[jnp.zeros((Sql, 1), jnp.float32) for _ in range(Hq)]
                for _ in range(B)]

        def process_block(origin):
            ki = kcol + origin * Skl
            mask = (jnp.abs(qrow - ki) <= 128) | (ki < 32) | (qrow < 32)
            for b in range(B):
                kb = kfull[b, pl.ds(origin * Skl, Skl), :]
                vb = vfull[b, pl.ds(origin * Skl, Skl), :]
                ks = ksc[b, pl.ds(origin * Skl, Skl)]
                vs = vsc[b, pl.ds(origin * Skl, Skl)]
                for h in range(Hq):
                    qh = q[b][:, h * Dh:(h + 1) * Dh]
                    kh = kb[:, h * Dh:(h + 1) * Dh].astype(jnp.bfloat16)
                    s = lax.dot_general(
                        qh, kh, (((1,), (1,)), ((), ())),
                        preferred_element_type=jnp.float32,
                    ) * ks[None, :]
                    p = jnp.exp(jnp.where(mask, s, -1e9))
                    lsum[b][h] = lsum[b][h] + jnp.sum(p, axis=-1,
                                                      keepdims=True)
                    acc[b][h] = acc[b][h] + jnp.dot(
                        (p * vs[None, :]).astype(jnp.bfloat16),
                        vb[:, h * Dh:(h + 1) * Dh].astype(jnp.bfloat16),
                        preferred_element_type=jnp.float32,
                    )

        process_block(my)
        for j, origin in enumerate((right, left, diag)):
            pltpu.make_async_remote_copy(
                src_ref=kfull.at[:, pl.ds(origin * Skl, Skl), :],
                dst_ref=kfull.at[:, pl.ds(origin * Skl, Skl), :],
                send_sem=ksend.at[j], recv_sem=krecv.at[j],
                device_id=(origin,), device_id_type=pl.DeviceIdType.MESH,
            ).wait_recv()
            pltpu.make_async_remote_copy(
                src_ref=vfull.at[:, pl.ds(origin * Skl, Skl), :],
                dst_ref=vfull.at[:, pl.ds(origin * Skl, Skl), :],
                send_sem=vsend.at[j], recv_sem=vrecv.at[j],
                device_id=(origin,), device_id_type=pl.DeviceIdType.MESH,
            ).wait_recv()
            for i, buf in enumerate((ksc, vsc)):
                pltpu.make_async_remote_copy(
                    src_ref=buf.at[:, pl.ds(origin * Skl, Skl)],
                    dst_ref=buf.at[:, pl.ds(origin * Skl, Skl)],
                    send_sem=ssend.at[i, j], recv_sem=srecv.at[i, j],
                    device_id=(origin,), device_id_type=pl.DeviceIdType.MESH,
                ).wait_recv()
            process_block(origin)

        for b in range(B):
            ctx = jnp.concatenate(
                [(acc[b][h] / lsum[b][h]).astype(jnp.bfloat16)
                 for h in range(Hq)],
                axis=1,
            )
            out_ref[b] = jnp.dot(ctx, wo,
                                 preferred_element_type=jnp.float32)

        for r in sends:
            r.wait_send()

    return pl.pallas_call(
        body,
        out_shape=jax.ShapeDtypeStruct((B, Sql, E), jnp.float32),
        in_specs=[
            pl.BlockSpec(memory_space=pltpu.VMEM),
            pl.BlockSpec(memory_space=pltpu.VMEM),
            pl.BlockSpec(memory_space=pl.ANY),
            pl.BlockSpec(memory_space=pl.ANY),
            pl.BlockSpec(memory_space=pltpu.VMEM),
        ],
        out_specs=pl.BlockSpec(memory_space=pltpu.VMEM),
        scratch_shapes=[
            pltpu.VMEM((B, Skv, HD), jnp.int8),
            pltpu.VMEM((B, Skv, HD), jnp.int8),
            pltpu.VMEM((B, Skv), jnp.float32),
            pltpu.VMEM((B, Skv), jnp.float32),
            pltpu.VMEM((B, Skl, Hq, Dh), jnp.float32),
            pltpu.VMEM((B, Skl, Hq, Dh), jnp.float32),
            pltpu.SemaphoreType.DMA((3,)),
            pltpu.SemaphoreType.DMA((3,)),
            pltpu.SemaphoreType.DMA((3,)),
            pltpu.SemaphoreType.DMA((3,)),
            pltpu.SemaphoreType.DMA((2, 3)),
            pltpu.SemaphoreType.DMA((2, 3)),
            pltpu.SemaphoreType.DMA((2,)),
        ],
        compiler_params=pltpu.CompilerParams(collective_id=0),
    )(x, Wq, K_ext, V_ext, Wo)
